# baseline (speedup 1.0000x reference)
import jax
import jax.numpy as jnp
from jax import lax
from jax.experimental import pallas as pl
from jax.experimental.pallas import tpu as pltpu
from jax.experimental.pallas import tpu_sc as plsc

N_NODES = 100000
EMBED = 128
MASK_NUM = 10000

COPY_ROWS = 20000

CHUNK = 64
N_FULL = MASK_NUM // CHUNK
TAIL = MASK_NUM - N_FULL * CHUNK
TAIL_OFF = N_FULL * CHUNK
N_TILES = 32
BASE_CH = N_FULL // N_TILES
EXTRA_FULL = N_FULL - BASE_CH * N_TILES
MAX_CH = BASE_CH + 1

_vector_mesh = plsc.VectorSubcoreMesh(core_axis_name="c", subcore_axis_name="s")


def _copy_body(x_ref, o_ref):
    o_ref[...] = x_ref[...]


def _tc_copy(embeds):
    return pl.pallas_call(
        _copy_body,
        grid=(N_NODES // COPY_ROWS,),
        in_specs=[pl.BlockSpec((COPY_ROWS, EMBED), lambda i: (i, 0))],
        out_specs=pl.BlockSpec((COPY_ROWS, EMBED), lambda i: (i, 0)),
        out_shape=jax.ShapeDtypeStruct((N_NODES, EMBED), jnp.float32),
    )(embeds)


def _sc_scatter(mask_token, seeds1d, out_ref):
    @pl.kernel(
        mesh=_vector_mesh,
        out_type=(),
        scratch_types=[
            pltpu.VMEM((MAX_CH * CHUNK, EMBED), jnp.float32),
            [pltpu.VMEM((CHUNK,), jnp.int32)] * MAX_CH,
            pltpu.VMEM((TAIL,), jnp.int32),
            pltpu.SemaphoreType.DMA((MAX_CH + 1,)),
        ],
    )
    def k(x_hbm, di_hbm, o_hbm, rows_v, dv, dt, sems):
        w = lax.axis_index("s") * 2 + lax.axis_index("c")

        def start_chunk(j):
            off = pl.multiple_of((w + N_TILES * j) * CHUNK, CHUNK)
            li = pltpu.async_copy(
                di_hbm.at[pl.ds(off, CHUNK)], dv[j], sems.at[j])
            lr = pltpu.async_copy(
                x_hbm.at[pl.ds(off, CHUNK)],
                rows_v.at[pl.ds(j * CHUNK, CHUNK)], sems.at[j])
            return li, lr

        def run(n, tail):
            ls = [start_chunk(j) for j in range(n)]
            if tail:
                ti = pltpu.async_copy(
                    di_hbm.at[pl.ds(TAIL_OFF, TAIL)], dt, sems.at[MAX_CH])
                tr = pltpu.async_copy(
                    x_hbm.at[pl.ds(TAIL_OFF, TAIL)],
                    rows_v.at[pl.ds(BASE_CH * CHUNK, TAIL)], sems.at[MAX_CH])
            ss = []
            for j in range(n):
                li, lr = ls[j]
                li.wait()
                lr.wait()
                ss.append(pltpu.async_copy(
                    rows_v.at[pl.ds(j * CHUNK, CHUNK)],
                    o_hbm.at[dv[j]], sems.at[j]))
            if tail:
                ti.wait()
                tr.wait()
                ss.append(pltpu.async_copy(
                    rows_v.at[pl.ds(BASE_CH * CHUNK, TAIL)],
                    o_hbm.at[dt], sems.at[MAX_CH]))
            for s in ss:
                s.wait()

        @pl.when(w < EXTRA_FULL)
        def _():
            run(MAX_CH, False)

        @pl.when(jnp.logical_and(w >= EXTRA_FULL, w < N_TILES - 1))
        def _():
            run(BASE_CH, False)

        @pl.when(w == N_TILES - 1)
        def _():
            run(BASE_CH, True)

    k(mask_token, seeds1d, out_ref)


CP_ROWS = 200
CP_CHUNKS = N_NODES // CP_ROWS
CP_BASE = CP_CHUNKS // N_TILES
CP_EXTRA = CP_CHUNKS - CP_BASE * N_TILES


def _sc_copy(embeds):
    @pl.kernel(
        mesh=_vector_mesh,
        out_type=jax.ShapeDtypeStruct((N_NODES, EMBED), jnp.float32),
        scratch_types=[
            pltpu.VMEM((CP_ROWS, EMBED), jnp.float32),
            pltpu.VMEM((CP_ROWS, EMBED), jnp.float32),
            pltpu.SemaphoreType.DMA((2,)),
            pltpu.SemaphoreType.DMA((2,)),
        ],
    )
    def k(x_hbm, o_hbm, b0, b1, rsem, wsem):
        w = lax.axis_index("s") * 2 + lax.axis_index("c")
        bufs = (b0, b1)

        def rd(i):
            off = pl.multiple_of((w + N_TILES * i) * CP_ROWS, CP_ROWS)
            return pltpu.async_copy(
                x_hbm.at[pl.ds(off, CP_ROWS)], bufs[i % 2], rsem.at[i % 2])

        def wr(i):
            off = pl.multiple_of((w + N_TILES * i) * CP_ROWS, CP_ROWS)
            return pltpu.async_copy(
                bufs[i % 2], o_hbm.at[pl.ds(off, CP_ROWS)], wsem.at[i % 2])

        def run(nch):
            r = [None] * nch
            wl = [None] * nch
            r[0] = rd(0)
            for i in range(nch):
                if i + 1 < nch:
                    if i - 1 >= 0:
                        wl[i - 1].wait()
                    r[i + 1] = rd(i + 1)
                r[i].wait()
                wl[i] = wr(i)
            wl[nch - 1].wait()
            if nch > 1:
                wl[nch - 2].wait()

        @pl.when(w < CP_EXTRA)
        def _():
            run(CP_BASE + 1)

        @pl.when(w >= CP_EXTRA)
        def _():
            run(CP_BASE)

    return k(embeds)


def kernel(embeds, mask_token, seeds):
    seeds_i = seeds.astype(jnp.int32)
    out_ref = jax.new_ref(_sc_copy(embeds))
    return jax.freeze(out_ref), seeds


# speedup vs baseline: 1.0598x; 1.0598x over previous
import jax
import jax.numpy as jnp
from jax import lax
from jax.experimental import pallas as pl
from jax.experimental.pallas import tpu as pltpu
from jax.experimental.pallas import tpu_sc as plsc

N_NODES = 100000
EMBED = 128
MASK_NUM = 10000

COPY_ROWS = 20000

CHUNK = 64
N_FULL = MASK_NUM // CHUNK
TAIL = MASK_NUM - N_FULL * CHUNK
TAIL_OFF = N_FULL * CHUNK
N_TILES = 32
BASE_CH = N_FULL // N_TILES
EXTRA_FULL = N_FULL - BASE_CH * N_TILES
MAX_CH = BASE_CH + 1

_vector_mesh = plsc.VectorSubcoreMesh(core_axis_name="c", subcore_axis_name="s")


def _copy_body(x_ref, o_ref):
    o_ref[...] = x_ref[...]


def _tc_copy(embeds):
    return pl.pallas_call(
        _copy_body,
        grid=(N_NODES // COPY_ROWS,),
        in_specs=[pl.BlockSpec((COPY_ROWS, EMBED), lambda i: (i, 0))],
        out_specs=pl.BlockSpec((COPY_ROWS, EMBED), lambda i: (i, 0)),
        out_shape=jax.ShapeDtypeStruct((N_NODES, EMBED), jnp.float32),
    )(embeds)


def _sc_scatter(mask_token, seeds1d, out_ref):
    @pl.kernel(
        mesh=_vector_mesh,
        out_type=(),
        scratch_types=[
            pltpu.VMEM((MAX_CH * CHUNK, EMBED), jnp.float32),
            [pltpu.VMEM((CHUNK,), jnp.int32)] * MAX_CH,
            pltpu.VMEM((TAIL,), jnp.int32),
            pltpu.SemaphoreType.DMA((MAX_CH + 1,)),
        ],
    )
    def k(x_hbm, di_hbm, o_hbm, rows_v, dv, dt, sems):
        w = lax.axis_index("s") * 2 + lax.axis_index("c")

        def start_chunk(j):
            off = pl.multiple_of((w + N_TILES * j) * CHUNK, CHUNK)
            li = pltpu.async_copy(
                di_hbm.at[pl.ds(off, CHUNK)], dv[j], sems.at[j])
            lr = pltpu.async_copy(
                x_hbm.at[pl.ds(off, CHUNK)],
                rows_v.at[pl.ds(j * CHUNK, CHUNK)], sems.at[j])
            return li, lr

        def run(n, tail):
            ls = [start_chunk(j) for j in range(n)]
            if tail:
                ti = pltpu.async_copy(
                    di_hbm.at[pl.ds(TAIL_OFF, TAIL)], dt, sems.at[MAX_CH])
                tr = pltpu.async_copy(
                    x_hbm.at[pl.ds(TAIL_OFF, TAIL)],
                    rows_v.at[pl.ds(BASE_CH * CHUNK, TAIL)], sems.at[MAX_CH])
            ss = []
            for j in range(n):
                li, lr = ls[j]
                li.wait()
                lr.wait()
                ss.append(pltpu.async_copy(
                    rows_v.at[pl.ds(j * CHUNK, CHUNK)],
                    o_hbm.at[dv[j]], sems.at[j]))
            if tail:
                ti.wait()
                tr.wait()
                ss.append(pltpu.async_copy(
                    rows_v.at[pl.ds(BASE_CH * CHUNK, TAIL)],
                    o_hbm.at[dt], sems.at[MAX_CH]))
            for s in ss:
                s.wait()

        @pl.when(w < EXTRA_FULL)
        def _():
            run(MAX_CH, False)

        @pl.when(jnp.logical_and(w >= EXTRA_FULL, w < N_TILES - 1))
        def _():
            run(BASE_CH, False)

        @pl.when(w == N_TILES - 1)
        def _():
            run(BASE_CH, True)

    k(mask_token, seeds1d, out_ref)


def kernel(embeds, mask_token, seeds):
    seeds_i = seeds.astype(jnp.int32)
    out_ref = jax.new_ref(_tc_copy(embeds))
    _sc_scatter(mask_token, seeds_i, out_ref)
    return jax.freeze(out_ref), seeds
